# baseline (speedup 1.0000x reference)
import functools

import jax
import jax.numpy as jnp
from jax import lax
from jax.experimental import pallas as pl
from jax.experimental.pallas import tpu as pltpu
from jax.experimental.pallas import tpu_sc as plsc

N = 10000
E = 320000
HID = 128
HALF = N // 2
NC = 2
NS = 16
L = 16
EC = E // NS
SUB = 2000
BATCH = 128
CAP = SUB + 2 * BATCH
CROWS = CAP // BATCH + 1
CNT_SEG = 5120
ACC_ROWS = 2 * HALF + 16
OUT_ROWS = 48

_mesh = plsc.VectorSubcoreMesh(
    core_axis_name="c", subcore_axis_name="s", num_cores=NC, num_subcores=NS)
_sc_params = pltpu.CompilerParams(needs_layout_passes=False)


def _phase_a_body(src_hbm, dst_hbm, xnt_hbm, emb_hbm,
                  sums_hbm, cntp_hbm, h_hbm,
                  xnt_v, src_v, dst_v, cxs_v, crow_v, cnt_v, gbuf,
                  acc_sh, sem0, sem1):
    c = lax.axis_index("c")
    s = lax.axis_index("s")
    cN = c * HALF
    iota = lax.iota(jnp.int32, L)
    zf = jnp.zeros((L,), jnp.float32)
    ones = jnp.ones((L,), jnp.float32)

    pltpu.sync_copy(xnt_hbm, xnt_v)

    def _zcnt(i, carry):
        for j in range(HID // L):
            cnt_v[i, pl.ds(j * L, L)] = zf
        return carry
    lax.fori_loop(0, CNT_SEG // 64, _zcnt, 0)

    def _zg(i, carry):
        for j in range(HID // L):
            gbuf[0, i, pl.ds(j * L, L)] = zf
        return carry
    lax.fori_loop(0, BATCH, _zg, 0)

    zbase = jnp.minimum(s * 632, ACC_ROWS - 640)

    def _zacc(i, carry):
        pltpu.sync_copy(gbuf.at[0], acc_sh.at[pl.ds(zbase + i * BATCH, BATCH)])
        return carry
    lax.fori_loop(0, 640 // BATCH, _zacc, 0)

    plsc.subcore_barrier()

    hstart = cN + jnp.minimum(s * 312, HALF - 320)

    def _hstage(g, carry):
        v = xnt_v[pl.ds(hstart + g * L, L)]
        cxs_v[pl.ds(g * L, L)] = lax.bitwise_and(v, 0x3FFF)
        return carry
    lax.fori_loop(0, 320 // L, _hstage, 0)
    for hoff in (0, 128, 192):
        pltpu.async_copy(emb_hbm.at[cxs_v.at[pl.ds(hoff, BATCH)]],
                         gbuf.at[0], sem0).wait()
        pltpu.sync_copy(gbuf.at[0],
                        h_hbm.at[pl.ds(hstart + hoff, BATCH)])

    eb = s * EC

    def _segment(seg, carry):
        sub_base = eb + seg * SUB
        pltpu.sync_copy(src_hbm.at[pl.ds(sub_base, SUB)], src_v)
        pltpu.sync_copy(dst_hbm.at[pl.ds(sub_base, SUB)], dst_v)

        def _grp(g, off):
            sg = src_v[pl.ds(g * L, L)]
            dg = dst_v[pl.ds(g * L, L)]
            vs = plsc.load_gather(xnt_v, [sg])
            vd = plsc.load_gather(xnt_v, [dg])
            ts = lax.shift_right_logical(vs, 30)
            td = lax.shift_right_logical(vd, 30)
            xs = lax.bitwise_and(vs, 0x3FFF)
            et = jnp.where(ts == td, 1, 0)
            owned = jnp.where(dg >= HALF, 1, 0) == c
            dl = dg - cN
            row = et * HALF + dl
            cf = et * CNT_SEG + dl
            plsc.addupdate_scatter(
                cnt_v,
                [lax.shift_right_logical(cf, 7), lax.bitwise_and(cf, 127)],
                ones, mask=owned)
            pos = off + plsc.cumsum(owned.astype(jnp.int32)) - 1
            plsc.store_scatter(cxs_v, [pos], xs, mask=owned)
            plsc.store_scatter(
                crow_v,
                [lax.shift_right_logical(pos, 7),
                 lax.bitwise_and(pos, BATCH - 1)],
                row, mask=owned)
            return off + plsc.all_reduce_population_count(owned)

        off = lax.fori_loop(0, SUB // L, _grp, jnp.zeros((L,), jnp.int32))

        for pg in range(BATCH // L):
            ppos = off + pg * L + iota
            plsc.store_scatter(cxs_v, [ppos], jnp.zeros((L,), jnp.int32))
            plsc.store_scatter(
                crow_v,
                [lax.shift_right_logical(ppos, 7),
                 lax.bitwise_and(ppos, BATCH - 1)],
                2 * HALF + iota)

        ntot = jnp.max(off)
        nbk = lax.div(ntot + (BATCH - 1), BATCH)

        def _fire(k, carry):
            pltpu.async_copy(
                emb_hbm.at[cxs_v.at[pl.ds(k * BATCH, BATCH)]],
                gbuf.at[0], sem0).wait()
            pltpu.sync_copy(gbuf.at[0], acc_sh.at[crow_v.at[k]], add=True)
            return carry

        lax.fori_loop(0, nbk, _fire, 0)
        return carry

    lax.fori_loop(0, EC // SUB, _segment, 0)

    plsc.subcore_barrier()

    pltpu.sync_copy(cnt_v, cntp_hbm.at[pl.ds((c * NS + s) * 80, 80)])
    et_s = lax.div(s, 8)
    jj = lax.rem(s, 8)
    dl0 = jnp.minimum(jj * 632, HALF - 632)
    pltpu.sync_copy(acc_sh.at[pl.ds(et_s * HALF + dl0, 632)],
                    sums_hbm.at[pl.ds(et_s * N + cN + dl0, 632)])


_phase_a = functools.partial(
    pl.kernel,
    out_type=(
        jax.ShapeDtypeStruct((2 * N, HID), jnp.float32),
        jax.ShapeDtypeStruct((NC * NS * 80, HID), jnp.float32),
        jax.ShapeDtypeStruct((N, HID), jnp.float32),
    ),
    mesh=_mesh,
    scratch_types=[
        pltpu.VMEM((N,), jnp.int32),
        pltpu.VMEM((SUB,), jnp.int32),
        pltpu.VMEM((SUB,), jnp.int32),
        pltpu.VMEM((CAP,), jnp.int32),
        pltpu.VMEM((CROWS, BATCH), jnp.int32),
        pltpu.VMEM((40 * 2, HID), jnp.float32),
        pltpu.VMEM((1, BATCH, HID), jnp.float32),
        pltpu.VMEM_SHARED((ACC_ROWS, HID), jnp.float32),
        pltpu.SemaphoreType.DMA,
        pltpu.SemaphoreType.DMA,
    ],
    compiler_params=_sc_params,
)(_phase_a_body)


def _cnt_reduce_body(c0_ref, c1_ref, o_ref):
    o_ref[0] = jnp.sum(c0_ref[...], axis=0)
    o_ref[1] = jnp.sum(c1_ref[...], axis=0)


def _cnt_reduce(c0p, c1p):
    return pl.pallas_call(
        _cnt_reduce_body,
        out_shape=jax.ShapeDtypeStruct((2, 2 * CNT_SEG), jnp.float32),
    )(c0p, c1p)


def _phase_b_body(sums_ref, cnt_ref, h_ref, ntf_ref, w0_ref, root0_ref,
                  b0_ref, wcat_ref, badd_ref, zpack_ref):
    c0 = cnt_ref[0]
    c1 = cnt_ref[1]
    r0 = 1.0 / jnp.maximum(c0, 1.0)
    r1 = 1.0 / jnp.maximum(c1, 1.0)
    m0 = sums_ref[0] * r0
    m1 = sums_ref[1] * r1
    h = h_ref[...]
    acc = jnp.dot(m0, w0_ref[0], preferred_element_type=jnp.float32)
    acc += jnp.dot(m1, w0_ref[1], preferred_element_type=jnp.float32)
    acc += jnp.dot(h, root0_ref[...], preferred_element_type=jnp.float32)
    h1 = jnp.maximum(acc + b0_ref[...], 0.0)
    zz = jnp.dot(h1, wcat_ref[...], preferred_element_type=jnp.float32)
    col = lax.broadcasted_iota(jnp.int32, (1, 8), 1)
    e3 = (col == 3).astype(jnp.float32)
    e4 = (col == 4).astype(jnp.float32)
    e5 = (col == 5).astype(jnp.float32)
    zpack_ref[...] = (zz + badd_ref[...] + r0 * e3 + r1 * e4
                      + ntf_ref[...] * e5)


def _phase_b(sums, cnt, h, ntf, w0, root0, b0, wcat, badd):
    bn = 1000
    return pl.pallas_call(
        _phase_b_body,
        grid=(N // bn,),
        in_specs=[
            pl.BlockSpec((2, bn, HID), lambda i: (0, i, 0)),
            pl.BlockSpec((2, bn, 1), lambda i: (0, i, 0)),
            pl.BlockSpec((bn, HID), lambda i: (i, 0)),
            pl.BlockSpec((bn, 1), lambda i: (i, 0)),
            pl.BlockSpec((2, HID, HID), lambda i: (0, 0, 0)),
            pl.BlockSpec((HID, HID), lambda i: (0, 0)),
            pl.BlockSpec((1, HID), lambda i: (0, 0)),
            pl.BlockSpec((HID, 8), lambda i: (0, 0)),
            pl.BlockSpec((1, 8), lambda i: (0, 0)),
        ],
        out_specs=pl.BlockSpec((bn, 8), lambda i: (i, 0)),
        out_shape=jax.ShapeDtypeStruct((N, 8), jnp.float32),
    )(sums, cnt, h, ntf, w0, root0, b0, wcat, badd)


def _phase_c_body(src_hbm, dst_hbm, zp_hbm, outp_hbm,
                  zp_v, src_v, dst_v, out_v, oiota_v, fbuf, out_sh):
    c = lax.axis_index("c")
    s = lax.axis_index("s")
    cN = c * HALF
    iota = lax.iota(jnp.int32, L)
    zf = jnp.zeros((L,), jnp.float32)

    pltpu.sync_copy(zp_hbm, zp_v)

    def _zo(i, carry):
        for j in range(HID // L):
            out_v[i, pl.ds(j * L, L)] = zf
        return carry
    lax.fori_loop(0, OUT_ROWS, _zo, 0)

    def _oiota(i, carry):
        oiota_v[pl.ds(i * L, L)] = i * L + iota
        return carry
    lax.fori_loop(0, OUT_ROWS // L, _oiota, 0)

    @pl.when(s == 0)
    def _():
        pltpu.sync_copy(out_v, out_sh)

    plsc.subcore_barrier()

    eb = s * EC

    def _segment(seg, carry):
        sub_base = eb + seg * SUB
        pltpu.sync_copy(src_hbm.at[pl.ds(sub_base, SUB)], src_v)
        pltpu.sync_copy(dst_hbm.at[pl.ds(sub_base, SUB)], dst_v)

        def _grp(g, carry2):
            sg = src_v[pl.ds(g * L, L)]
            dg = dst_v[pl.ds(g * L, L)]
            s8 = sg * 8
            d8 = dg * 8
            ts = plsc.load_gather(zp_v, [s8 + 5])
            td = plsc.load_gather(zp_v, [d8 + 5])
            et = jnp.where(ts == td, 1, 0)
            owned = jnp.where(dg >= HALF, 1, 0) == c
            dl = dg - cN
            zsrc = plsc.load_gather(zp_v, [s8 + et])
            rdst = plsc.load_gather(
                zp_v, [jnp.where(owned, d8, 0) + 3 + et])
            val = zsrc * rdst
            plsc.addupdate_scatter(
                out_v,
                [lax.shift_right_logical(dl, 7), lax.bitwise_and(dl, 127)],
                val, mask=owned)
            return carry2

        lax.fori_loop(0, SUB // L, _grp, 0)
        return carry

    lax.fori_loop(0, EC // SUB, _segment, 0)

    pltpu.sync_copy(out_v, out_sh.at[oiota_v], add=True)

    plsc.subcore_barrier()

    rb = jnp.minimum(s, 4) * 8
    pltpu.sync_copy(out_sh.at[pl.ds(rb, 8)], fbuf)
    for r in range(8):
        for j in range(HID // L):
            dl = (rb + r) * HID + j * L + iota
            valid = dl < HALF
            nsafe = jnp.where(valid, cN + dl, 0)
            zb = plsc.load_gather(zp_v, [nsafe * 8 + 2])
            fbuf[r, pl.ds(j * L, L)] = (
                fbuf[r, pl.ds(j * L, L)] + jnp.where(valid, zb, 0.0))
    pltpu.sync_copy(fbuf, outp_hbm.at[pl.ds(c * OUT_ROWS + rb, 8)])


_phase_c = functools.partial(
    pl.kernel,
    out_type=jax.ShapeDtypeStruct((NC * OUT_ROWS, HID), jnp.float32),
    mesh=_mesh,
    scratch_types=[
        pltpu.VMEM((N * 8,), jnp.float32),
        pltpu.VMEM((SUB,), jnp.int32),
        pltpu.VMEM((SUB,), jnp.int32),
        pltpu.VMEM((OUT_ROWS, HID), jnp.float32),
        pltpu.VMEM((OUT_ROWS,), jnp.int32),
        pltpu.VMEM((8, HID), jnp.float32),
        pltpu.VMEM_SHARED((OUT_ROWS, HID), jnp.float32),
    ],
    compiler_params=_sc_params,
)(_phase_c_body)


def kernel(x, edge_index, node_type, emb, w0, root0, b0, w1, root1, b1):
    x = x.astype(jnp.int32)
    src = edge_index[0].astype(jnp.int32)
    dst = edge_index[1].astype(jnp.int32)
    node_type = node_type.astype(jnp.int32)
    xnt = jnp.bitwise_or(x, jnp.left_shift(node_type, 30))

    sums_flat, cntp, h = _phase_a(src, dst, xnt, emb)
    sums = sums_flat.reshape(2, N, HID)

    cp = cntp.reshape(NC, NS, 2 * CNT_SEG)
    cnt2 = _cnt_reduce(cp[0], cp[1])
    cnt0 = jnp.concatenate([cnt2[0, :HALF], cnt2[1, :HALF]])
    cnt1 = jnp.concatenate([cnt2[0, CNT_SEG:CNT_SEG + HALF],
                            cnt2[1, CNT_SEG:CNT_SEG + HALF]])
    cnt = jnp.stack([cnt0, cnt1])[:, :, None]

    wcat = jnp.concatenate(
        [w1[0], w1[1], root1, jnp.zeros((HID, 5), jnp.float32)], axis=1)
    badd = (b1[0] * (jnp.arange(8) == 2).astype(jnp.float32))[None, :]
    ntf = node_type.astype(jnp.float32)[:, None]

    zpack = _phase_b(sums, cnt, h, ntf, w0, root0, b0.reshape(1, HID),
                     wcat, badd)

    outp = _phase_c(src, dst, zpack.reshape(-1))
    o = outp.reshape(NC, OUT_ROWS * HID)
    out = jnp.concatenate([o[0, :HALF], o[1, :HALF]])[:, None]
    return out


# speedup vs baseline: 1.3428x; 1.3428x over previous
import functools

import jax
import jax.numpy as jnp
from jax import lax
from jax.experimental import pallas as pl
from jax.experimental.pallas import tpu as pltpu
from jax.experimental.pallas import tpu_sc as plsc

N = 10000
E = 320000
HID = 128
HALF = N // 2
NC = 2
NS = 16
L = 16
EC = E // NS
SUB = 2000
BATCH = 64
CAP = SUB + 2 * BATCH
CROWS = CAP // BATCH + 1
CNT_SEG = 5120
ACC_ROWS = 2 * HALF + 16
OUT_ROWS = 48

_mesh = plsc.VectorSubcoreMesh(
    core_axis_name="c", subcore_axis_name="s", num_cores=NC, num_subcores=NS)
_sc_params = pltpu.CompilerParams(needs_layout_passes=False)


def _phase_a_body(src_hbm, dst_hbm, xnt_hbm, emb_hbm,
                  sums_hbm, cntp_hbm, h_hbm,
                  xnt_v, src_v, dst_v, cxs_v, crow_v, cnt_v, gbuf,
                  acc_sh, sem0, sem1):
    c = lax.axis_index("c")
    s = lax.axis_index("s")
    cN = c * HALF
    iota = lax.iota(jnp.int32, L)
    zf = jnp.zeros((L,), jnp.float32)
    ones = jnp.ones((L,), jnp.float32)

    pltpu.sync_copy(xnt_hbm, xnt_v)

    def _zcnt(i, carry):
        for j in range(HID // L):
            cnt_v[i, pl.ds(j * L, L)] = zf
        return carry
    lax.fori_loop(0, CNT_SEG // 64, _zcnt, 0)

    def _zg(i, carry):
        for j in range(HID // L):
            gbuf[0, i, pl.ds(j * L, L)] = zf
        return carry
    lax.fori_loop(0, BATCH, _zg, 0)

    zbase = jnp.minimum(s * 632, ACC_ROWS - 640)

    def _zacc(i, carry):
        pltpu.sync_copy(gbuf.at[0], acc_sh.at[pl.ds(zbase + i * BATCH, BATCH)])
        return carry
    lax.fori_loop(0, 640 // BATCH, _zacc, 0)

    plsc.subcore_barrier()

    hstart = cN + jnp.minimum(s * 312, HALF - 320)

    def _hstage(g, carry):
        v = xnt_v[pl.ds(hstart + g * L, L)]
        cxs_v[pl.ds(g * L, L)] = lax.bitwise_and(v, 0x3FFF)
        return carry
    lax.fori_loop(0, 320 // L, _hstage, 0)
    for hk in range(5):
        pltpu.async_copy(emb_hbm.at[cxs_v.at[pl.ds(hk * BATCH, BATCH)]],
                         gbuf.at[0], sem0).wait()
        pltpu.sync_copy(gbuf.at[0],
                        h_hbm.at[pl.ds(hstart + hk * BATCH, BATCH)])

    eb = s * EC

    def _segment(seg, carry):
        sub_base = eb + seg * SUB
        pltpu.sync_copy(src_hbm.at[pl.ds(sub_base, SUB)], src_v)
        pltpu.sync_copy(dst_hbm.at[pl.ds(sub_base, SUB)], dst_v)

        def _grp(g, off):
            sg = src_v[pl.ds(g * L, L)]
            dg = dst_v[pl.ds(g * L, L)]
            vs = plsc.load_gather(xnt_v, [sg])
            vd = plsc.load_gather(xnt_v, [dg])
            ts = lax.shift_right_logical(vs, 30)
            td = lax.shift_right_logical(vd, 30)
            xs = lax.bitwise_and(vs, 0x3FFF)
            et = jnp.where(ts == td, 1, 0)
            owned = jnp.where(dg >= HALF, 1, 0) == c
            dl = dg - cN
            row = et * HALF + dl
            cf = et * CNT_SEG + dl
            plsc.addupdate_scatter(
                cnt_v,
                [lax.shift_right_logical(cf, 7), lax.bitwise_and(cf, 127)],
                ones, mask=owned)
            pos = off + plsc.cumsum(owned.astype(jnp.int32)) - 1
            plsc.store_scatter(cxs_v, [pos], xs, mask=owned)
            plsc.store_scatter(
                crow_v,
                [lax.shift_right_logical(pos, 6),
                 lax.bitwise_and(pos, BATCH - 1)],
                row, mask=owned)
            return off + plsc.all_reduce_population_count(owned)

        off = plsc.parallel_loop(
            0, SUB // L, carry=jnp.zeros((L,), jnp.int32), unroll=4)(_grp)

        for pg in range(BATCH // L):
            ppos = off + pg * L + iota
            plsc.store_scatter(cxs_v, [ppos], jnp.zeros((L,), jnp.int32))
            plsc.store_scatter(
                crow_v,
                [lax.shift_right_logical(ppos, 6),
                 lax.bitwise_and(ppos, BATCH - 1)],
                2 * HALF + iota)

        ntot = jnp.max(off)
        nbk = lax.div(ntot + (BATCH - 1), BATCH)

        @pl.when(nbk > 0)
        def _():
            pltpu.async_copy(emb_hbm.at[cxs_v.at[pl.ds(0, BATCH)]],
                             gbuf.at[0], sem0)

        @pl.when(nbk > 1)
        def _():
            pltpu.async_copy(emb_hbm.at[cxs_v.at[pl.ds(BATCH, BATCH)]],
                             gbuf.at[1], sem1)

        def _fire(k, carry):
            @pl.when(lax.rem(k, 2) == 0)
            def _():
                pltpu.make_async_copy(
                    emb_hbm.at[pl.ds(0, BATCH)], gbuf.at[0], sem0).wait()
                pltpu.sync_copy(gbuf.at[0], acc_sh.at[crow_v.at[k]], add=True)

                @pl.when(k + 2 < nbk)
                def _():
                    pltpu.async_copy(
                        emb_hbm.at[cxs_v.at[pl.ds((k + 2) * BATCH, BATCH)]],
                        gbuf.at[0], sem0)

            @pl.when(lax.rem(k, 2) == 1)
            def _():
                pltpu.make_async_copy(
                    emb_hbm.at[pl.ds(0, BATCH)], gbuf.at[1], sem1).wait()
                pltpu.sync_copy(gbuf.at[1], acc_sh.at[crow_v.at[k]], add=True)

                @pl.when(k + 2 < nbk)
                def _():
                    pltpu.async_copy(
                        emb_hbm.at[cxs_v.at[pl.ds((k + 2) * BATCH, BATCH)]],
                        gbuf.at[1], sem1)
            return carry

        lax.fori_loop(0, nbk, _fire, 0)
        return carry

    lax.fori_loop(0, EC // SUB, _segment, 0)

    plsc.subcore_barrier()

    pltpu.sync_copy(cnt_v, cntp_hbm.at[pl.ds((c * NS + s) * 80, 80)])
    et_s = lax.div(s, 8)
    jj = lax.rem(s, 8)
    dl0 = jnp.minimum(jj * 632, HALF - 632)
    pltpu.sync_copy(acc_sh.at[pl.ds(et_s * HALF + dl0, 632)],
                    sums_hbm.at[pl.ds(et_s * N + cN + dl0, 632)])


_phase_a = functools.partial(
    pl.kernel,
    out_type=(
        jax.ShapeDtypeStruct((2 * N, HID), jnp.float32),
        jax.ShapeDtypeStruct((NC * NS * 80, HID), jnp.float32),
        jax.ShapeDtypeStruct((N, HID), jnp.float32),
    ),
    mesh=_mesh,
    scratch_types=[
        pltpu.VMEM((N,), jnp.int32),
        pltpu.VMEM((SUB,), jnp.int32),
        pltpu.VMEM((SUB,), jnp.int32),
        pltpu.VMEM((CAP,), jnp.int32),
        pltpu.VMEM((CROWS, BATCH), jnp.int32),
        pltpu.VMEM((40 * 2, HID), jnp.float32),
        pltpu.VMEM((2, BATCH, HID), jnp.float32),
        pltpu.VMEM_SHARED((ACC_ROWS, HID), jnp.float32),
        pltpu.SemaphoreType.DMA,
        pltpu.SemaphoreType.DMA,
    ],
    compiler_params=_sc_params,
)(_phase_a_body)


def _cnt_reduce_body(c0_ref, c1_ref, o_ref):
    o_ref[0] = jnp.sum(c0_ref[...], axis=0)
    o_ref[1] = jnp.sum(c1_ref[...], axis=0)


def _cnt_reduce(c0p, c1p):
    return pl.pallas_call(
        _cnt_reduce_body,
        out_shape=jax.ShapeDtypeStruct((2, 2 * CNT_SEG), jnp.float32),
    )(c0p, c1p)


def _phase_b_body(sums_ref, cnt_ref, h_ref, ntf_ref, w0_ref, root0_ref,
                  b0_ref, wcat_ref, badd_ref, zpack_ref):
    c0 = cnt_ref[0]
    c1 = cnt_ref[1]
    r0 = 1.0 / jnp.maximum(c0, 1.0)
    r1 = 1.0 / jnp.maximum(c1, 1.0)
    m0 = sums_ref[0] * r0
    m1 = sums_ref[1] * r1
    h = h_ref[...]
    acc = jnp.dot(m0, w0_ref[0], preferred_element_type=jnp.float32)
    acc += jnp.dot(m1, w0_ref[1], preferred_element_type=jnp.float32)
    acc += jnp.dot(h, root0_ref[...], preferred_element_type=jnp.float32)
    h1 = jnp.maximum(acc + b0_ref[...], 0.0)
    zz = jnp.dot(h1, wcat_ref[...], preferred_element_type=jnp.float32)
    col = lax.broadcasted_iota(jnp.int32, (1, 8), 1)
    e3 = (col == 3).astype(jnp.float32)
    e4 = (col == 4).astype(jnp.float32)
    e5 = (col == 5).astype(jnp.float32)
    zpack_ref[...] = (zz + badd_ref[...] + r0 * e3 + r1 * e4
                      + ntf_ref[...] * e5)


def _phase_b(sums, cnt, h, ntf, w0, root0, b0, wcat, badd):
    bn = 1000
    return pl.pallas_call(
        _phase_b_body,
        grid=(N // bn,),
        in_specs=[
            pl.BlockSpec((2, bn, HID), lambda i: (0, i, 0)),
            pl.BlockSpec((2, bn, 1), lambda i: (0, i, 0)),
            pl.BlockSpec((bn, HID), lambda i: (i, 0)),
            pl.BlockSpec((bn, 1), lambda i: (i, 0)),
            pl.BlockSpec((2, HID, HID), lambda i: (0, 0, 0)),
            pl.BlockSpec((HID, HID), lambda i: (0, 0)),
            pl.BlockSpec((1, HID), lambda i: (0, 0)),
            pl.BlockSpec((HID, 8), lambda i: (0, 0)),
            pl.BlockSpec((1, 8), lambda i: (0, 0)),
        ],
        out_specs=pl.BlockSpec((bn, 8), lambda i: (i, 0)),
        out_shape=jax.ShapeDtypeStruct((N, 8), jnp.float32),
    )(sums, cnt, h, ntf, w0, root0, b0, wcat, badd)


def _phase_c_body(src_hbm, dst_hbm, zp_hbm, outp_hbm,
                  zp_v, src_v, dst_v, out_v, oiota_v, fbuf, out_sh):
    c = lax.axis_index("c")
    s = lax.axis_index("s")
    cN = c * HALF
    iota = lax.iota(jnp.int32, L)
    zf = jnp.zeros((L,), jnp.float32)

    pltpu.sync_copy(zp_hbm, zp_v)

    def _zo(i, carry):
        for j in range(HID // L):
            out_v[i, pl.ds(j * L, L)] = zf
        return carry
    lax.fori_loop(0, OUT_ROWS, _zo, 0)

    def _oiota(i, carry):
        oiota_v[pl.ds(i * L, L)] = i * L + iota
        return carry
    lax.fori_loop(0, OUT_ROWS // L, _oiota, 0)

    @pl.when(s == 0)
    def _():
        pltpu.sync_copy(out_v, out_sh)

    plsc.subcore_barrier()

    eb = s * EC

    def _segment(seg, carry):
        sub_base = eb + seg * SUB
        pltpu.sync_copy(src_hbm.at[pl.ds(sub_base, SUB)], src_v)
        pltpu.sync_copy(dst_hbm.at[pl.ds(sub_base, SUB)], dst_v)

        def _grp(g, carry2):
            sg = src_v[pl.ds(g * L, L)]
            dg = dst_v[pl.ds(g * L, L)]
            s8 = sg * 8
            d8 = dg * 8
            ts = plsc.load_gather(zp_v, [s8 + 5])
            td = plsc.load_gather(zp_v, [d8 + 5])
            et = jnp.where(ts == td, 1, 0)
            owned = jnp.where(dg >= HALF, 1, 0) == c
            dl = dg - cN
            zsrc = plsc.load_gather(zp_v, [s8 + et])
            rdst = plsc.load_gather(
                zp_v, [jnp.where(owned, d8, 0) + 3 + et])
            val = zsrc * rdst
            plsc.addupdate_scatter(
                out_v,
                [lax.shift_right_logical(dl, 7), lax.bitwise_and(dl, 127)],
                val, mask=owned)
            return carry2

        plsc.parallel_loop(0, SUB // L, carry=jnp.int32(0), unroll=4)(_grp)
        return carry

    lax.fori_loop(0, EC // SUB, _segment, 0)

    pltpu.sync_copy(out_v, out_sh.at[oiota_v], add=True)

    plsc.subcore_barrier()

    rb = jnp.minimum(s, 4) * 8
    pltpu.sync_copy(out_sh.at[pl.ds(rb, 8)], fbuf)
    for r in range(8):
        for j in range(HID // L):
            dl = (rb + r) * HID + j * L + iota
            valid = dl < HALF
            nsafe = jnp.where(valid, cN + dl, 0)
            zb = plsc.load_gather(zp_v, [nsafe * 8 + 2])
            fbuf[r, pl.ds(j * L, L)] = (
                fbuf[r, pl.ds(j * L, L)] + jnp.where(valid, zb, 0.0))
    pltpu.sync_copy(fbuf, outp_hbm.at[pl.ds(c * OUT_ROWS + rb, 8)])


_phase_c = functools.partial(
    pl.kernel,
    out_type=jax.ShapeDtypeStruct((NC * OUT_ROWS, HID), jnp.float32),
    mesh=_mesh,
    scratch_types=[
        pltpu.VMEM((N * 8,), jnp.float32),
        pltpu.VMEM((SUB,), jnp.int32),
        pltpu.VMEM((SUB,), jnp.int32),
        pltpu.VMEM((OUT_ROWS, HID), jnp.float32),
        pltpu.VMEM((OUT_ROWS,), jnp.int32),
        pltpu.VMEM((8, HID), jnp.float32),
        pltpu.VMEM_SHARED((OUT_ROWS, HID), jnp.float32),
    ],
    compiler_params=_sc_params,
)(_phase_c_body)


def kernel(x, edge_index, node_type, emb, w0, root0, b0, w1, root1, b1):
    x = x.astype(jnp.int32)
    src = edge_index[0].astype(jnp.int32)
    dst = edge_index[1].astype(jnp.int32)
    node_type = node_type.astype(jnp.int32)
    xnt = jnp.bitwise_or(x, jnp.left_shift(node_type, 30))

    sums_flat, cntp, h = _phase_a(src, dst, xnt, emb)
    sums = sums_flat.reshape(2, N, HID)

    cp = cntp.reshape(NC, NS, 2 * CNT_SEG)
    cnt2 = _cnt_reduce(cp[0], cp[1])
    cnt0 = jnp.concatenate([cnt2[0, :HALF], cnt2[1, :HALF]])
    cnt1 = jnp.concatenate([cnt2[0, CNT_SEG:CNT_SEG + HALF],
                            cnt2[1, CNT_SEG:CNT_SEG + HALF]])
    cnt = jnp.stack([cnt0, cnt1])[:, :, None]

    wcat = jnp.concatenate(
        [w1[0], w1[1], root1, jnp.zeros((HID, 5), jnp.float32)], axis=1)
    badd = (b1[0] * (jnp.arange(8) == 2).astype(jnp.float32))[None, :]
    ntf = node_type.astype(jnp.float32)[:, None]

    zpack = _phase_b(sums, cnt, h, ntf, w0, root0, b0.reshape(1, HID),
                     wcat, badd)

    outp = _phase_c(src, dst, zpack.reshape(-1))
    o = outp.reshape(NC, OUT_ROWS * HID)
    out = jnp.concatenate([o[0, :HALF], o[1, :HALF]])[:, None]
    return out


# speedup vs baseline: 1.3506x; 1.0058x over previous
import functools

import jax
import jax.numpy as jnp
from jax import lax
from jax.experimental import pallas as pl
from jax.experimental.pallas import tpu as pltpu
from jax.experimental.pallas import tpu_sc as plsc

N = 10000
E = 320000
HID = 128
HALF = N // 2
NC = 2
NS = 16
L = 16
EC = E // NS
SUB = 2000
BATCH = 64
CAP = SUB + 2 * BATCH
CROWS = CAP // BATCH + 1
CNT_SEG = 5120
ACC_ROWS = 2 * HALF + 16
OUT_ROWS = 48

_mesh = plsc.VectorSubcoreMesh(
    core_axis_name="c", subcore_axis_name="s", num_cores=NC, num_subcores=NS)
_sc_params = pltpu.CompilerParams(needs_layout_passes=False)


def _phase_a_body(src_hbm, dst_hbm, xnt_hbm, emb_hbm,
                  sums_hbm, cntp_hbm, h_hbm,
                  xnt_v, src_v, dst_v, cxs_v, crow_v, cnt_v, gbuf,
                  acc_sh, sem0, sem1):
    c = lax.axis_index("c")
    s = lax.axis_index("s")
    cN = c * HALF
    iota = lax.iota(jnp.int32, L)
    zf = jnp.zeros((L,), jnp.float32)
    ones = jnp.ones((L,), jnp.float32)

    pltpu.sync_copy(xnt_hbm, xnt_v)

    def _zcnt(i, carry):
        for j in range(HID // L):
            cnt_v[i, pl.ds(j * L, L)] = zf
        return carry
    lax.fori_loop(0, CNT_SEG // 64, _zcnt, 0)

    def _zg(i, carry):
        for j in range(HID // L):
            gbuf[0, i, pl.ds(j * L, L)] = zf
        return carry
    lax.fori_loop(0, BATCH, _zg, 0)

    zbase = jnp.minimum(s * 632, ACC_ROWS - 640)

    def _zacc(i, carry):
        pltpu.async_copy(gbuf.at[0],
                         acc_sh.at[pl.ds(zbase + i * BATCH, BATCH)], sem1)
        return carry
    lax.fori_loop(0, 640 // BATCH, _zacc, 0)

    def _zdrain(i, carry):
        pltpu.make_async_copy(
            gbuf.at[0], acc_sh.at[pl.ds(zbase, BATCH)], sem1).wait()
        return carry
    lax.fori_loop(0, 640 // BATCH, _zdrain, 0)

    hstart = cN + jnp.minimum(s * 312, HALF - 320)

    def _hstage(g, carry):
        v = xnt_v[pl.ds(hstart + g * L, L)]
        cxs_v[pl.ds(g * L, L)] = lax.bitwise_and(v, 0x3FFF)
        return carry
    lax.fori_loop(0, 320 // L, _hstage, 0)
    for hk in range(5):
        pltpu.async_copy(emb_hbm.at[cxs_v.at[pl.ds(hk * BATCH, BATCH)]],
                         gbuf.at[hk % 2], sem0 if hk % 2 == 0 else sem1)
        if hk > 0:
            pltpu.make_async_copy(
                emb_hbm.at[pl.ds(0, BATCH)], gbuf.at[(hk - 1) % 2],
                sem0 if (hk - 1) % 2 == 0 else sem1).wait()
            pltpu.sync_copy(
                gbuf.at[(hk - 1) % 2],
                h_hbm.at[pl.ds(hstart + (hk - 1) * BATCH, BATCH)])
    pltpu.make_async_copy(emb_hbm.at[pl.ds(0, BATCH)], gbuf.at[0],
                          sem0).wait()
    pltpu.sync_copy(gbuf.at[0], h_hbm.at[pl.ds(hstart + 4 * BATCH, BATCH)])

    plsc.subcore_barrier()

    eb = s * EC

    def _segment(seg, carry):
        sub_base = eb + seg * SUB
        pltpu.sync_copy(src_hbm.at[pl.ds(sub_base, SUB)], src_v)
        pltpu.sync_copy(dst_hbm.at[pl.ds(sub_base, SUB)], dst_v)

        def _grp(g, off):
            sg = src_v[pl.ds(g * L, L)]
            dg = dst_v[pl.ds(g * L, L)]
            vs = plsc.load_gather(xnt_v, [sg])
            vd = plsc.load_gather(xnt_v, [dg])
            ts = lax.shift_right_logical(vs, 30)
            td = lax.shift_right_logical(vd, 30)
            xs = lax.bitwise_and(vs, 0x3FFF)
            et = jnp.where(ts == td, 1, 0)
            owned = jnp.where(dg >= HALF, 1, 0) == c
            dl = dg - cN
            row = et * HALF + dl
            cf = et * CNT_SEG + dl
            plsc.addupdate_scatter(
                cnt_v,
                [lax.shift_right_logical(cf, 7), lax.bitwise_and(cf, 127)],
                ones, mask=owned)
            pos = off + plsc.cumsum(owned.astype(jnp.int32)) - 1
            plsc.store_scatter(cxs_v, [pos], xs, mask=owned)
            plsc.store_scatter(
                crow_v,
                [lax.shift_right_logical(pos, 6),
                 lax.bitwise_and(pos, BATCH - 1)],
                row, mask=owned)
            return off + plsc.all_reduce_population_count(owned)

        off = plsc.parallel_loop(
            0, SUB // L, carry=jnp.zeros((L,), jnp.int32), unroll=4)(_grp)

        for pg in range(BATCH // L):
            ppos = off + pg * L + iota
            plsc.store_scatter(cxs_v, [ppos], jnp.zeros((L,), jnp.int32))
            plsc.store_scatter(
                crow_v,
                [lax.shift_right_logical(ppos, 6),
                 lax.bitwise_and(ppos, BATCH - 1)],
                2 * HALF + iota)

        ntot = jnp.max(off)
        nbk = lax.div(ntot + (BATCH - 1), BATCH)

        @pl.when(nbk > 0)
        def _():
            pltpu.async_copy(emb_hbm.at[cxs_v.at[pl.ds(0, BATCH)]],
                             gbuf.at[0], sem0)

        @pl.when(nbk > 1)
        def _():
            pltpu.async_copy(emb_hbm.at[cxs_v.at[pl.ds(BATCH, BATCH)]],
                             gbuf.at[1], sem1)

        def _fire(k, carry):
            @pl.when(lax.rem(k, 2) == 0)
            def _():
                pltpu.make_async_copy(
                    emb_hbm.at[pl.ds(0, BATCH)], gbuf.at[0], sem0).wait()
                pltpu.sync_copy(gbuf.at[0], acc_sh.at[crow_v.at[k]], add=True)

                @pl.when(k + 2 < nbk)
                def _():
                    pltpu.async_copy(
                        emb_hbm.at[cxs_v.at[pl.ds((k + 2) * BATCH, BATCH)]],
                        gbuf.at[0], sem0)

            @pl.when(lax.rem(k, 2) == 1)
            def _():
                pltpu.make_async_copy(
                    emb_hbm.at[pl.ds(0, BATCH)], gbuf.at[1], sem1).wait()
                pltpu.sync_copy(gbuf.at[1], acc_sh.at[crow_v.at[k]], add=True)

                @pl.when(k + 2 < nbk)
                def _():
                    pltpu.async_copy(
                        emb_hbm.at[cxs_v.at[pl.ds((k + 2) * BATCH, BATCH)]],
                        gbuf.at[1], sem1)
            return carry

        lax.fori_loop(0, nbk, _fire, 0)
        return carry

    lax.fori_loop(0, EC // SUB, _segment, 0)

    plsc.subcore_barrier()

    pltpu.sync_copy(cnt_v, cntp_hbm.at[pl.ds((c * NS + s) * 80, 80)])
    et_s = lax.div(s, 8)
    jj = lax.rem(s, 8)
    dl0 = jnp.minimum(jj * 632, HALF - 632)
    pltpu.sync_copy(acc_sh.at[pl.ds(et_s * HALF + dl0, 632)],
                    sums_hbm.at[pl.ds(et_s * N + cN + dl0, 632)])


_phase_a = functools.partial(
    pl.kernel,
    out_type=(
        jax.ShapeDtypeStruct((2 * N, HID), jnp.float32),
        jax.ShapeDtypeStruct((NC * NS * 80, HID), jnp.float32),
        jax.ShapeDtypeStruct((N, HID), jnp.float32),
    ),
    mesh=_mesh,
    scratch_types=[
        pltpu.VMEM((N,), jnp.int32),
        pltpu.VMEM((SUB,), jnp.int32),
        pltpu.VMEM((SUB,), jnp.int32),
        pltpu.VMEM((CAP,), jnp.int32),
        pltpu.VMEM((CROWS, BATCH), jnp.int32),
        pltpu.VMEM((40 * 2, HID), jnp.float32),
        pltpu.VMEM((2, BATCH, HID), jnp.float32),
        pltpu.VMEM_SHARED((ACC_ROWS, HID), jnp.float32),
        pltpu.SemaphoreType.DMA,
        pltpu.SemaphoreType.DMA,
    ],
    compiler_params=_sc_params,
)(_phase_a_body)


def _cnt_reduce_body(c0_ref, c1_ref, o_ref):
    o_ref[0] = jnp.sum(c0_ref[...], axis=0)
    o_ref[1] = jnp.sum(c1_ref[...], axis=0)


def _cnt_reduce(c0p, c1p):
    return pl.pallas_call(
        _cnt_reduce_body,
        out_shape=jax.ShapeDtypeStruct((2, 2 * CNT_SEG), jnp.float32),
    )(c0p, c1p)


def _phase_b_body(sums_ref, cnt_ref, h_ref, ntf_ref, w0_ref, root0_ref,
                  b0_ref, wcat_ref, badd_ref, zpack_ref):
    c0 = cnt_ref[0]
    c1 = cnt_ref[1]
    r0 = 1.0 / jnp.maximum(c0, 1.0)
    r1 = 1.0 / jnp.maximum(c1, 1.0)
    m0 = sums_ref[0] * r0
    m1 = sums_ref[1] * r1
    h = h_ref[...]
    acc = jnp.dot(m0, w0_ref[0], preferred_element_type=jnp.float32)
    acc += jnp.dot(m1, w0_ref[1], preferred_element_type=jnp.float32)
    acc += jnp.dot(h, root0_ref[...], preferred_element_type=jnp.float32)
    h1 = jnp.maximum(acc + b0_ref[...], 0.0)
    zz = jnp.dot(h1, wcat_ref[...], preferred_element_type=jnp.float32)
    col = lax.broadcasted_iota(jnp.int32, (1, 8), 1)
    e3 = (col == 3).astype(jnp.float32)
    e4 = (col == 4).astype(jnp.float32)
    e5 = (col == 5).astype(jnp.float32)
    zpack_ref[...] = (zz + badd_ref[...] + r0 * e3 + r1 * e4
                      + ntf_ref[...] * e5)


def _phase_b(sums, cnt, h, ntf, w0, root0, b0, wcat, badd):
    bn = 1000
    return pl.pallas_call(
        _phase_b_body,
        grid=(N // bn,),
        in_specs=[
            pl.BlockSpec((2, bn, HID), lambda i: (0, i, 0)),
            pl.BlockSpec((2, bn, 1), lambda i: (0, i, 0)),
            pl.BlockSpec((bn, HID), lambda i: (i, 0)),
            pl.BlockSpec((bn, 1), lambda i: (i, 0)),
            pl.BlockSpec((2, HID, HID), lambda i: (0, 0, 0)),
            pl.BlockSpec((HID, HID), lambda i: (0, 0)),
            pl.BlockSpec((1, HID), lambda i: (0, 0)),
            pl.BlockSpec((HID, 8), lambda i: (0, 0)),
            pl.BlockSpec((1, 8), lambda i: (0, 0)),
        ],
        out_specs=pl.BlockSpec((bn, 8), lambda i: (i, 0)),
        out_shape=jax.ShapeDtypeStruct((N, 8), jnp.float32),
    )(sums, cnt, h, ntf, w0, root0, b0, wcat, badd)


def _phase_c_body(src_hbm, dst_hbm, zp_hbm, outp_hbm,
                  zp_v, src_v, dst_v, out_v, oiota_v, fbuf, out_sh):
    c = lax.axis_index("c")
    s = lax.axis_index("s")
    cN = c * HALF
    iota = lax.iota(jnp.int32, L)
    zf = jnp.zeros((L,), jnp.float32)

    pltpu.sync_copy(zp_hbm, zp_v)

    def _zo(i, carry):
        for j in range(HID // L):
            out_v[i, pl.ds(j * L, L)] = zf
        return carry
    lax.fori_loop(0, OUT_ROWS, _zo, 0)

    def _oiota(i, carry):
        oiota_v[pl.ds(i * L, L)] = i * L + iota
        return carry
    lax.fori_loop(0, OUT_ROWS // L, _oiota, 0)

    @pl.when(s == 0)
    def _():
        pltpu.sync_copy(out_v, out_sh)

    plsc.subcore_barrier()

    eb = s * EC

    def _segment(seg, carry):
        sub_base = eb + seg * SUB
        pltpu.sync_copy(src_hbm.at[pl.ds(sub_base, SUB)], src_v)
        pltpu.sync_copy(dst_hbm.at[pl.ds(sub_base, SUB)], dst_v)

        def _grp(g, carry2):
            sg = src_v[pl.ds(g * L, L)]
            dg = dst_v[pl.ds(g * L, L)]
            s8 = sg * 8
            d8 = dg * 8
            ts = plsc.load_gather(zp_v, [s8 + 5])
            td = plsc.load_gather(zp_v, [d8 + 5])
            et = jnp.where(ts == td, 1, 0)
            owned = jnp.where(dg >= HALF, 1, 0) == c
            dl = dg - cN
            zsrc = plsc.load_gather(zp_v, [s8 + et])
            rdst = plsc.load_gather(
                zp_v, [jnp.where(owned, d8, 0) + 3 + et])
            val = zsrc * rdst
            plsc.addupdate_scatter(
                out_v,
                [lax.shift_right_logical(dl, 7), lax.bitwise_and(dl, 127)],
                val, mask=owned)
            return carry2

        plsc.parallel_loop(0, SUB // L, carry=jnp.int32(0), unroll=4)(_grp)
        return carry

    lax.fori_loop(0, EC // SUB, _segment, 0)

    pltpu.sync_copy(out_v, out_sh.at[oiota_v], add=True)

    plsc.subcore_barrier()

    rb = jnp.minimum(s, 4) * 8
    pltpu.sync_copy(out_sh.at[pl.ds(rb, 8)], fbuf)
    for r in range(8):
        for j in range(HID // L):
            dl = (rb + r) * HID + j * L + iota
            valid = dl < HALF
            nsafe = jnp.where(valid, cN + dl, 0)
            zb = plsc.load_gather(zp_v, [nsafe * 8 + 2])
            fbuf[r, pl.ds(j * L, L)] = (
                fbuf[r, pl.ds(j * L, L)] + jnp.where(valid, zb, 0.0))
    pltpu.sync_copy(fbuf, outp_hbm.at[pl.ds(c * OUT_ROWS + rb, 8)])


_phase_c = functools.partial(
    pl.kernel,
    out_type=jax.ShapeDtypeStruct((NC * OUT_ROWS, HID), jnp.float32),
    mesh=_mesh,
    scratch_types=[
        pltpu.VMEM((N * 8,), jnp.float32),
        pltpu.VMEM((SUB,), jnp.int32),
        pltpu.VMEM((SUB,), jnp.int32),
        pltpu.VMEM((OUT_ROWS, HID), jnp.float32),
        pltpu.VMEM((OUT_ROWS,), jnp.int32),
        pltpu.VMEM((8, HID), jnp.float32),
        pltpu.VMEM_SHARED((OUT_ROWS, HID), jnp.float32),
    ],
    compiler_params=_sc_params,
)(_phase_c_body)


def kernel(x, edge_index, node_type, emb, w0, root0, b0, w1, root1, b1):
    x = x.astype(jnp.int32)
    src = edge_index[0].astype(jnp.int32)
    dst = edge_index[1].astype(jnp.int32)
    node_type = node_type.astype(jnp.int32)
    xnt = jnp.bitwise_or(x, jnp.left_shift(node_type, 30))

    sums_flat, cntp, h = _phase_a(src, dst, xnt, emb)
    sums = sums_flat.reshape(2, N, HID)

    cp = cntp.reshape(NC, NS, 2 * CNT_SEG)
    cnt2 = _cnt_reduce(cp[0], cp[1])
    cnt0 = jnp.concatenate([cnt2[0, :HALF], cnt2[1, :HALF]])
    cnt1 = jnp.concatenate([cnt2[0, CNT_SEG:CNT_SEG + HALF],
                            cnt2[1, CNT_SEG:CNT_SEG + HALF]])
    cnt = jnp.stack([cnt0, cnt1])[:, :, None]

    wcat = jnp.concatenate(
        [w1[0], w1[1], root1, jnp.zeros((HID, 5), jnp.float32)], axis=1)
    badd = (b1[0] * (jnp.arange(8) == 2).astype(jnp.float32))[None, :]
    ntf = node_type.astype(jnp.float32)[:, None]

    zpack = _phase_b(sums, cnt, h, ntf, w0, root0, b0.reshape(1, HID),
                     wcat, badd)

    outp = _phase_c(src, dst, zpack.reshape(-1))
    o = outp.reshape(NC, OUT_ROWS * HID)
    out = jnp.concatenate([o[0, :HALF], o[1, :HALF]])[:, None]
    return out
